# Initial kernel scaffold; baseline (speedup 1.0000x reference)
import jax
import jax.numpy as jnp
from jax.experimental import pallas as pl


def kernel(x, edge_index, W, b, a):
    raise NotImplementedError("write your pallas kernel here")


# speedup vs baseline: 11.3066x; 11.3066x over previous
import functools

import jax
import jax.numpy as jnp
from jax import lax
from jax.experimental import pallas as pl
from jax.experimental.pallas import tpu as pltpu
from jax.experimental.pallas import tpu_sc as plsc

N = 10000
D = 128
NP = 10240
NW = 32
NSUB = 16
CH = 79
CL = 128
EP = NW * CH * CL
RPS = NP // NSUB


def _sc_mesh():
    return plsc.VectorSubcoreMesh(core_axis_name="c", subcore_axis_name="s")


def _deg_sc(src3, dst3, zeros1):
    @functools.partial(
        pl.kernel,
        out_type=[jax.ShapeDtypeStruct((2, NP), jnp.float32),
                  jax.ShapeDtypeStruct((NW, CH, CL), jnp.int32)],
        scratch_types=[pltpu.VMEM((CH, CL), jnp.int32),
                       pltpu.VMEM((CH, CL), jnp.int32),
                       pltpu.VMEM((CH, CL), jnp.float32),
                       pltpu.VMEM_SHARED((NP,), jnp.float32)],
        mesh=_sc_mesh(),
    )
    def k(src_h, dst_h, z_h, degp_h, dstp_h, src_v, dst_v, keep_v, deg_acc):
        c = lax.axis_index("c")
        s = lax.axis_index("s")
        w = s * 2 + c
        pltpu.sync_copy(src_h.at[w], src_v)
        pltpu.sync_copy(dst_h.at[w], dst_v)
        pltpu.sync_copy(z_h.at[pl.ds(s * RPS, RPS)],
                        deg_acc.at[pl.ds(s * RPS, RPS)])
        plsc.subcore_barrier()

        def body(j, carry):
            for c8 in range(8):
                sl = pl.ds(c8 * 16, 16)
                sv = src_v[j, sl]
                dv = dst_v[j, sl]
                eq = sv == dv
                keep_v[j, sl] = jnp.where(eq, 0.0, 1.0)
                dst_v[j, sl] = jnp.where(eq, N, dv)
            pltpu.sync_copy(keep_v.at[j], deg_acc.at[src_v.at[j]], add=True)
            return carry

        lax.fori_loop(0, CH, body, 0)
        plsc.subcore_barrier()
        pltpu.sync_copy(deg_acc.at[pl.ds(s * RPS, RPS)],
                        degp_h.at[c, pl.ds(s * RPS, RPS)])
        pltpu.sync_copy(dst_v, dstp_h.at[w])

    return k(src3, dst3, zeros1)


def _hop_sc(g, src3, dstp3, zeros2):
    @functools.partial(
        pl.kernel,
        out_type=jax.ShapeDtypeStruct((2, NP, D), jnp.float32),
        scratch_types=[pltpu.VMEM((CH, CL), jnp.int32),
                       pltpu.VMEM((CH, CL), jnp.int32),
                       pltpu.VMEM((CL, D), jnp.float32),
                       pltpu.VMEM_SHARED((NP, D), jnp.float32),
                       pltpu.SemaphoreType.DMA],
        mesh=_sc_mesh(),
    )
    def k(g_h, src_h, dstp_h, z_h, acc_h, src_v, dst_v, rowbuf, acc, sem):
        c = lax.axis_index("c")
        s = lax.axis_index("s")
        w = s * 2 + c
        pltpu.sync_copy(src_h.at[w], src_v)
        pltpu.sync_copy(dstp_h.at[w], dst_v)
        pltpu.sync_copy(z_h, acc.at[pl.ds(s * RPS, RPS)])
        plsc.subcore_barrier()

        def body(j, carry):
            pltpu.async_copy(g_h.at[src_v.at[j]], rowbuf, sem).wait()
            pltpu.sync_copy(rowbuf, acc.at[dst_v.at[j]], add=True)
            return carry

        lax.fori_loop(0, CH, body, 0)
        plsc.subcore_barrier()
        pltpu.sync_copy(acc.at[pl.ds(s * RPS, RPS)],
                        acc_h.at[c, pl.ds(s * RPS, RPS)])

    return k(g, src3, dstp3, zeros2)


def _prep_tc(deg_p, x_pad):
    R = 512
    grid = NP // R

    def body(dp_ref, x_ref, dis_ref, g_ref):
        deg = dp_ref[0] + dp_ref[1] + 1.0
        dis = lax.rsqrt(deg)
        dis_ref[...] = dis
        g_ref[...] = x_ref[...] * dis

    return pl.pallas_call(
        body,
        grid=(grid,),
        in_specs=[pl.BlockSpec((2, R, 1), lambda i: (0, i, 0)),
                  pl.BlockSpec((R, D), lambda i: (i, 0))],
        out_specs=[pl.BlockSpec((R, 1), lambda i: (i, 0)),
                   pl.BlockSpec((R, D), lambda i: (i, 0))],
        out_shape=[jax.ShapeDtypeStruct((NP, 1), jnp.float32),
                   jax.ShapeDtypeStruct((NP, D), jnp.float32)],
    )(deg_p, x_pad)


def _mid_tc(accs, g1, dis):
    R = 512
    grid = NP // R

    def body(a_ref, g_ref, dis_ref, h_ref, g2_ref):
        dis_b = dis_ref[...]
        h1 = (a_ref[0] + a_ref[1] + g_ref[...]) * dis_b
        h_ref[...] = h1
        g2_ref[...] = h1 * dis_b

    return pl.pallas_call(
        body,
        grid=(grid,),
        in_specs=[pl.BlockSpec((2, R, D), lambda i: (0, i, 0)),
                  pl.BlockSpec((R, D), lambda i: (i, 0)),
                  pl.BlockSpec((R, 1), lambda i: (i, 0))],
        out_specs=[pl.BlockSpec((R, D), lambda i: (i, 0)),
                   pl.BlockSpec((R, D), lambda i: (i, 0))],
        out_shape=[jax.ShapeDtypeStruct((NP, D), jnp.float32),
                   jax.ShapeDtypeStruct((NP, D), jnp.float32)],
    )(accs, g1, dis)


def _final_tc(accs, g2, dis, x_pad, h1, W, b2, a2):
    R = 400
    grid = N // R

    def body(a_ref, g_ref, dis_ref, x_ref, h1_ref, w_ref, b_ref, s_ref, o_ref):
        h2 = (a_ref[0] + a_ref[1] + g_ref[...]) * dis_ref[...]
        acc = jnp.dot(x_ref[...], w_ref[0:128, :],
                      preferred_element_type=jnp.float32)
        acc = acc + jnp.dot(h1_ref[...], w_ref[128:256, :],
                            preferred_element_type=jnp.float32)
        acc = acc + jnp.dot(h2, w_ref[256:384, :],
                            preferred_element_type=jnp.float32)
        acc = acc + b_ref[...]
        slope = s_ref[0, 0]
        o_ref[...] = jnp.where(acc > 0, acc, slope * acc)

    return pl.pallas_call(
        body,
        grid=(grid,),
        in_specs=[pl.BlockSpec((2, R, D), lambda i: (0, i, 0)),
                  pl.BlockSpec((R, D), lambda i: (i, 0)),
                  pl.BlockSpec((R, 1), lambda i: (i, 0)),
                  pl.BlockSpec((R, D), lambda i: (i, 0)),
                  pl.BlockSpec((R, D), lambda i: (i, 0)),
                  pl.BlockSpec((3 * D, D), lambda i: (0, 0)),
                  pl.BlockSpec((1, D), lambda i: (0, 0)),
                  pl.BlockSpec((1, 1), lambda i: (0, 0))],
        out_specs=pl.BlockSpec((R, D), lambda i: (i, 0)),
        out_shape=jax.ShapeDtypeStruct((N, D), jnp.float32),
    )(accs, g2, dis, x_pad, h1, W, b2, a2)


def kernel(x, edge_index, W, b, a):
    E = edge_index.shape[1]
    pad = EP - E
    src = edge_index[0]
    dst = edge_index[1]
    zpad = jnp.zeros((pad,), jnp.int32)
    src3 = jnp.concatenate([src, zpad]).reshape(NW, CH, CL)
    dst3 = jnp.concatenate([dst, zpad]).reshape(NW, CH, CL)
    x_pad = jnp.pad(x, ((0, NP - N), (0, 0)))
    zeros1 = jnp.zeros((NP,), jnp.float32)
    zeros2 = jnp.zeros((RPS, D), jnp.float32)

    deg_p, dstp3 = _deg_sc(src3, dst3, zeros1)
    dis, g1 = _prep_tc(deg_p.reshape(2, NP, 1), x_pad)
    acc1 = _hop_sc(g1, src3, dstp3, zeros2)
    h1, g2 = _mid_tc(acc1, g1, dis)
    acc2 = _hop_sc(g2, src3, dstp3, zeros2)
    out = _final_tc(acc2, g2, dis, x_pad, h1, W,
                    b.reshape(1, D), a.reshape(1, 1))
    return out
